# baseline (speedup 1.0000x reference)
import functools

import jax
import jax.numpy as jnp
from jax import lax
from jax.experimental import pallas as pl
from jax.experimental.pallas import tpu as pltpu
from jax.experimental.pallas import tpu_sc as plsc

B = 16384
D = 32
H = 128
NC = 2
NS = 16
NW = NC * NS
BPW = B // NW


def _sc_gather_body(u_idx, i_idx, l_idx, c_idx, ue, ie, le, ce, out,
                    idx0, idx1, idx2, idx3, rows0, rows1, rows2, rows3, sem):
    wid = lax.axis_index("s") * NC + lax.axis_index("c")
    base = wid * BPW
    idxs = (u_idx, i_idx, l_idx, c_idx)
    tables = (ue, ie, le, ce)
    idx_bufs = (idx0, idx1, idx2, idx3)
    row_bufs = (rows0, rows1, rows2, rows3)
    for t in range(4):
        pltpu.sync_copy(idxs[t].at[pl.ds(base, BPW)], idx_bufs[t])
    copies = [
        pltpu.async_copy(tables[t].at[idx_bufs[t]], row_bufs[t], sem)
        for t in range(4)
    ]
    for cp in copies:
        cp.wait()
    for t in range(4):
        pltpu.sync_copy(row_bufs[t], out.at[t, pl.ds(base, BPW)])


@functools.partial(jax.jit, static_argnames=())
def _sc_gather(user, item, language, category, ue, ie, le, ce):
    mesh = plsc.VectorSubcoreMesh(core_axis_name="c", subcore_axis_name="s")
    scratch = (
        [pltpu.VMEM((BPW,), jnp.int32) for _ in range(4)]
        + [pltpu.VMEM((BPW, D), jnp.float32) for _ in range(4)]
        + [pltpu.SemaphoreType.DMA]
    )
    k = pl.kernel(
        _sc_gather_body,
        out_type=jax.ShapeDtypeStruct((4, B, D), jnp.float32),
        mesh=mesh,
        scratch_types=scratch,
        compiler_params=pltpu.CompilerParams(use_tc_tiling_on_sc=False),
    )
    return k(user, item, language, category, ue, ie, le, ce)


BM = 2048


def _mlp_body(g_ref, w1_ref, b1_ref, w2_ref, b2_ref, out_ref):
    h = jnp.dot(g_ref[0], w1_ref[0:32, :], preferred_element_type=jnp.float32)
    h = h + jnp.dot(g_ref[1], w1_ref[32:64, :], preferred_element_type=jnp.float32)
    h = h + jnp.dot(g_ref[2], w1_ref[64:96, :], preferred_element_type=jnp.float32)
    h = h + jnp.dot(g_ref[3], w1_ref[96:128, :], preferred_element_type=jnp.float32)
    h = jnp.maximum(h + b1_ref[0, :][None, :], 0.0)
    out_ref[...] = (
        jnp.sum(h * w2_ref[0, :][None, :], axis=1, keepdims=True) + b2_ref[0, 0]
    )


def _mlp(g, W1, b1, W2, b2):
    w2_row = W2.reshape(1, H)
    b1_row = b1.reshape(1, H)
    b2_s = b2.reshape(1, 1)
    out = pl.pallas_call(
        _mlp_body,
        grid=(B // BM,),
        in_specs=[
            pl.BlockSpec((4, BM, D), lambda i: (0, i, 0)),
            pl.BlockSpec((H, H), lambda i: (0, 0)),
            pl.BlockSpec((1, H), lambda i: (0, 0)),
            pl.BlockSpec((1, H), lambda i: (0, 0)),
            pl.BlockSpec((1, 1), lambda i: (0, 0)),
        ],
        out_specs=pl.BlockSpec((BM, 1), lambda i: (i, 0)),
        out_shape=jax.ShapeDtypeStruct((B, 1), jnp.float32),
    )(g, W1, b1_row, w2_row, b2_s)
    return out[:, 0]


def kernel(user, item, language, category,
           user_emb, item_emb, language_emb, category_emb,
           W1, b1, W2, b2):
    user = user.astype(jnp.int32)
    item = item.astype(jnp.int32)
    language = language.astype(jnp.int32)
    category = category.astype(jnp.int32)
    g = _sc_gather(user, item, language, category,
                   user_emb, item_emb, language_emb, category_emb)
    return _mlp(g, W1, b1, W2, b2)
